# baseline (speedup 1.0000x reference)
import functools

import jax
import jax.numpy as jnp
from jax import lax
from jax.experimental import pallas as pl
from jax.experimental.pallas import tpu as pltpu
from jax.experimental.pallas import tpu_sc as plsc

N = 10000
E = 160000
D = 256
H = 256
HH = 128
B = 64
C = 10
NB = 10
R = N // NB

NSC = 2
NT = 16
EC = 128
EPT = E // NT
CPT = 80
EPTP = CPT * EC
BCH = 16
NBATCH = CPT // BCH
RPT = N // NT


ZR = 624
ZREM = N - (NT - 1) * ZR - ZR


def _sc_agg_body(h_hbm, src_hbm, dst_hbm, zeros_hbm, out_hbm,
                 sidx_v, didx_v, rows_v, acc, sem, sem2):
    c = lax.axis_index("c")
    s = lax.axis_index("s")
    pltpu.sync_copy(zeros_hbm.at[pl.ds(s * ZR, ZR)], acc.at[pl.ds(s * ZR, ZR)])

    @pl.when(s == NT - 1)
    def _zrem():
        pltpu.sync_copy(zeros_hbm.at[pl.ds(NT * ZR, ZREM)],
                        acc.at[pl.ds(NT * ZR, ZREM)])

    tid = c * NT + s
    pltpu.sync_copy(src_hbm.at[tid, pl.ds(0, BCH)], sidx_v.at[0])
    pltpu.sync_copy(dst_hbm.at[s, pl.ds(0, BCH)], didx_v.at[0])
    pltpu.sync_copy(src_hbm.at[tid, pl.ds(BCH, BCH)], sidx_v.at[1])
    pltpu.sync_copy(dst_hbm.at[s, pl.ds(BCH, BCH)], didx_v.at[1])
    plsc.subcore_barrier()

    def chunk(ci, carry):
        m = ci // BCH
        b = lax.rem(m, 2)
        r = ci - m * BCH

        @pl.when((r == 0) & (ci >= BCH) & (m + 1 < NBATCH))
        def _stage():
            nb = m + 1
            sl = lax.rem(nb, 2)
            pltpu.sync_copy(src_hbm.at[tid, pl.ds(nb * BCH, BCH)], sidx_v.at[sl])
            pltpu.sync_copy(dst_hbm.at[s, pl.ds(nb * BCH, BCH)], didx_v.at[sl])

        pltpu.async_copy(h_hbm.at[sidx_v.at[b, r]], rows_v.at[0], sem).wait()
        pltpu.sync_copy(rows_v.at[0], acc.at[didx_v.at[b, r]], add=True)
        return carry

    lax.fori_loop(0, CPT, chunk, 0)
    plsc.subcore_barrier()
    pltpu.sync_copy(acc.at[pl.ds(s * ZR, ZR)],
                    out_hbm.at[pl.ds(c * N + s * ZR, ZR)])

    @pl.when(s == NT - 1)
    def _wrem():
        pltpu.sync_copy(acc.at[pl.ds(NT * ZR, ZREM)],
                        out_hbm.at[pl.ds(c * N + NT * ZR, ZREM)])


@functools.cache
def _sc_agg_call():
    return pl.kernel(
        _sc_agg_body,
        out_type=jax.ShapeDtypeStruct((2 * N, HH), jnp.float32),
        mesh=plsc.VectorSubcoreMesh(core_axis_name="c", subcore_axis_name="s",
                                    num_cores=NSC, num_subcores=NT),
        scratch_types=[
            pltpu.VMEM((2, BCH, EC), jnp.int32),
            pltpu.VMEM((2, BCH, EC), jnp.int32),
            pltpu.VMEM((2, EC, HH), jnp.float32),
            pltpu.VMEM_SHARED((N + 8, HH), jnp.float32),
            pltpu.SemaphoreType.DMA,
            pltpu.SemaphoreType.DMA,
        ],
    )


def _gin_mlp_body(eps_ref, h0, h1, a0, a1, w1, b1, w2, b2, u_ref, st_ref):
    ep = 1.0 + eps_ref[0, 0]
    z0 = ep * h0[...] + a0[...]
    z1 = ep * h1[...] + a1[...]
    t = jnp.dot(z0, w1[0:HH, :], preferred_element_type=jnp.float32)
    t = t + jnp.dot(z1, w1[HH:D, :], preferred_element_type=jnp.float32)
    t = jnp.maximum(t + b1[...], 0.0)
    u = jnp.dot(t, w2[...], preferred_element_type=jnp.float32) + b2[...]
    u = jnp.maximum(u, 0.0)
    u_ref[...] = u
    stats = jnp.concatenate([jnp.sum(u, axis=0, keepdims=True),
                             jnp.sum(u * u, axis=0, keepdims=True)], axis=0)

    @pl.when(pl.program_id(0) == 0)
    def _init():
        st_ref[...] = stats

    @pl.when(pl.program_id(0) > 0)
    def _acc():
        st_ref[...] += stats


_gin_mlp = pl.pallas_call(
    _gin_mlp_body,
    grid=(NB,),
    in_specs=[
        pl.BlockSpec(memory_space=pltpu.SMEM),
        pl.BlockSpec((R, HH), lambda i: (i, 0)),
        pl.BlockSpec((R, HH), lambda i: (i + NB, 0)),
        pl.BlockSpec((R, HH), lambda i: (i, 0)),
        pl.BlockSpec((R, HH), lambda i: (i + NB, 0)),
        pl.BlockSpec((D, H), lambda i: (0, 0)),
        pl.BlockSpec((1, H), lambda i: (0, 0)),
        pl.BlockSpec((H, H), lambda i: (0, 0)),
        pl.BlockSpec((1, H), lambda i: (0, 0)),
    ],
    out_specs=[
        pl.BlockSpec((R, H), lambda i: (i, 0)),
        pl.BlockSpec((2, H), lambda i: (0, 0)),
    ],
    out_shape=[
        jax.ShapeDtypeStruct((N, H), jnp.float32),
        jax.ShapeDtypeStruct((2, H), jnp.float32),
    ],
)


def _bn_pool_body(u, st, gam, bet, bat, hout_ref, g_ref):
    m = st[0:1, :] * (1.0 / N)
    var = st[1:2, :] * (1.0 / N) - m * m
    a = gam[...] * lax.rsqrt(var + 1e-5)
    b = bet[...] - m * a
    y = u[...] * a + b
    hout_ref[0] = y[:, 0:HH]
    hout_ref[1] = y[:, HH:H]
    seg = lax.broadcasted_iota(jnp.int32, (B, R), 0)
    msk = jnp.where(bat[0] == seg, 1.0, 0.0)
    gp = jnp.dot(msk, y, preferred_element_type=jnp.float32)

    @pl.when(pl.program_id(0) == 0)
    def _init():
        g_ref[...] = gp

    @pl.when(pl.program_id(0) > 0)
    def _acc():
        g_ref[...] += gp


_bn_pool = pl.pallas_call(
    _bn_pool_body,
    grid=(NB,),
    in_specs=[
        pl.BlockSpec((R, H), lambda i: (i, 0)),
        pl.BlockSpec((2, H), lambda i: (0, 0)),
        pl.BlockSpec((1, H), lambda i: (0, 0)),
        pl.BlockSpec((1, H), lambda i: (0, 0)),
        pl.BlockSpec((1, 1, R), lambda i: (i, 0, 0)),
    ],
    out_specs=[
        pl.BlockSpec((2, R, HH), lambda i: (0, i, 0)),
        pl.BlockSpec((B, H), lambda i: (0, 0)),
    ],
    out_shape=[
        jax.ShapeDtypeStruct((2, N, HH), jnp.float32),
        jax.ShapeDtypeStruct((B, H), jnp.float32),
    ],
)


def _cls_body(g1, g2, g3, g4, bat, w1, b1, w2, b2, w3, b3, w4, b4, out_ref):
    seg = lax.broadcasted_iota(jnp.int32, (B, R), 0)
    cnt = jnp.zeros((B, 1), jnp.float32)
    for r in range(NB):
        msk = jnp.where(bat[r:r + 1, :] == seg, 1.0, 0.0)
        cnt = cnt + jnp.sum(msk, axis=1, keepdims=True)
    inv = 1.0 / jnp.maximum(cnt, 1.0)
    z = jnp.dot(g1[...] * inv, w1[0:H, :], preferred_element_type=jnp.float32)
    z = z + jnp.dot(g2[...] * inv, w1[H:2 * H, :], preferred_element_type=jnp.float32)
    z = z + jnp.dot(g3[...] * inv, w1[2 * H:3 * H, :], preferred_element_type=jnp.float32)
    z = z + jnp.dot(g4[...] * inv, w1[3 * H:4 * H, :], preferred_element_type=jnp.float32)
    z = jnp.maximum(z + b1[...], 0.0)
    z = jnp.maximum(jnp.dot(z, w2[...], preferred_element_type=jnp.float32) + b2[...], 0.0)
    z = jnp.maximum(jnp.dot(z, w3[...], preferred_element_type=jnp.float32) + b3[...], 0.0)
    z = jnp.dot(z, w4[...], preferred_element_type=jnp.float32) + b4[...]
    mx = jnp.max(z, axis=1, keepdims=True)
    e = z - mx
    out_ref[...] = e - jnp.log(jnp.sum(jnp.exp(e), axis=1, keepdims=True))


_cls = pl.pallas_call(
    _cls_body,
    grid=(1,),
    in_specs=[
        pl.BlockSpec((B, H), lambda i: (0, 0)),
        pl.BlockSpec((B, H), lambda i: (0, 0)),
        pl.BlockSpec((B, H), lambda i: (0, 0)),
        pl.BlockSpec((B, H), lambda i: (0, 0)),
        pl.BlockSpec((NB, R), lambda i: (0, 0)),
        pl.BlockSpec((4 * H, 2 * H), lambda i: (0, 0)),
        pl.BlockSpec((1, 2 * H), lambda i: (0, 0)),
        pl.BlockSpec((2 * H, H), lambda i: (0, 0)),
        pl.BlockSpec((1, H), lambda i: (0, 0)),
        pl.BlockSpec((H, H), lambda i: (0, 0)),
        pl.BlockSpec((1, H), lambda i: (0, 0)),
        pl.BlockSpec((H, C), lambda i: (0, 0)),
        pl.BlockSpec((1, C), lambda i: (0, 0)),
    ],
    out_specs=pl.BlockSpec((B, C), lambda i: (0, 0)),
    out_shape=jax.ShapeDtypeStruct((B, C), jnp.float32),
)


def kernel(x, edge_index, batch, params):
    src = edge_index[0]
    dst = edge_index[1]
    src2 = src.reshape(NT, EPT)
    srcp = jnp.pad(src2, ((0, 0), (0, EPTP - EPT)))
    srcoff = jnp.concatenate([srcp, srcp + N]).reshape(2 * NT, CPT, EC)
    dstp = jnp.pad(dst.reshape(NT, EPT), ((0, 0), (0, EPTP - EPT)),
                   constant_values=N)
    dst3d = dstp.reshape(NT, CPT, EC)
    zeros_h = jnp.zeros((N, HH), jnp.float32)
    bat2d = batch.reshape(NB, R)
    bat3d = batch.reshape(NB, 1, R)
    h_cat = jnp.concatenate([x[:, :HH], x[:, HH:]], axis=0)

    gs = []
    for p in params['convs']:
        agg = _sc_agg_call()(h_cat, srcoff, dst3d, zeros_h)
        eps = jnp.reshape(p['eps'], (1, 1))
        u, st = _gin_mlp(eps, h_cat, h_cat, agg, agg,
                         p['W1'], p['b1'].reshape(1, H),
                         p['W2'], p['b2'].reshape(1, H))
        hout, g = _bn_pool(u, st, p['gamma'].reshape(1, H),
                           p['beta'].reshape(1, H), bat3d)
        h_cat = hout.reshape(2 * N, HH)
        gs.append(g)

    cl = params['cls']
    return _cls(gs[0], gs[1], gs[2], gs[3], bat2d,
                cl[0]['W'], cl[0]['b'].reshape(1, 2 * H),
                cl[1]['W'], cl[1]['b'].reshape(1, H),
                cl[2]['W'], cl[2]['b'].reshape(1, H),
                cl[3]['W'], cl[3]['b'].reshape(1, C))


# speedup vs baseline: 1.4445x; 1.4445x over previous
import functools

import jax
import jax.numpy as jnp
from jax import lax
from jax.experimental import pallas as pl
from jax.experimental.pallas import tpu as pltpu
from jax.experimental.pallas import tpu_sc as plsc

N = 10000
E = 160000
D = 256
H = 256
HH = 128
B = 64
C = 10
NB = 10
R = N // NB

NSC = 2
NT = 16
EC = 80
EPT = E // NT
CPT = EPT // EC
RPT = N // NT


ZR = 624
ZREM = N - (NT - 1) * ZR - ZR


def _sc_agg_body(h_hbm, src_hbm, dst_hbm, zeros_hbm, out_hbm,
                 sidx_v, didx_v, rows_v, acc, sem, sem2):
    c = lax.axis_index("c")
    s = lax.axis_index("s")
    pltpu.sync_copy(zeros_hbm.at[pl.ds(s * ZR, ZR)], acc.at[pl.ds(s * ZR, ZR)])

    @pl.when(s == NT - 1)
    def _zrem():
        pltpu.sync_copy(zeros_hbm.at[pl.ds(NT * ZR, ZREM)],
                        acc.at[pl.ds(NT * ZR, ZREM)])

    pltpu.sync_copy(src_hbm.at[c * NT + s], sidx_v)
    pltpu.sync_copy(dst_hbm.at[s], didx_v)
    plsc.subcore_barrier()

    def chunk(ci, carry):
        pltpu.async_copy(h_hbm.at[sidx_v.at[ci]], rows_v.at[0], sem).wait()
        pltpu.sync_copy(rows_v.at[0], acc.at[didx_v.at[ci]], add=True)
        return carry

    lax.fori_loop(0, CPT, chunk, 0)
    plsc.subcore_barrier()
    pltpu.sync_copy(acc.at[pl.ds(s * ZR, ZR)],
                    out_hbm.at[pl.ds(c * N + s * ZR, ZR)])

    @pl.when(s == NT - 1)
    def _wrem():
        pltpu.sync_copy(acc.at[pl.ds(NT * ZR, ZREM)],
                        out_hbm.at[pl.ds(c * N + NT * ZR, ZREM)])


@functools.cache
def _sc_agg_call():
    return pl.kernel(
        _sc_agg_body,
        out_type=jax.ShapeDtypeStruct((2 * N, HH), jnp.float32),
        mesh=plsc.VectorSubcoreMesh(core_axis_name="c", subcore_axis_name="s",
                                    num_cores=NSC, num_subcores=NT),
        scratch_types=[
            pltpu.VMEM((CPT, EC), jnp.int32),
            pltpu.VMEM((CPT, EC), jnp.int32),
            pltpu.VMEM((1, EC, HH), jnp.float32),
            pltpu.VMEM_SHARED((N, HH), jnp.float32),
            pltpu.SemaphoreType.DMA,
            pltpu.SemaphoreType.DMA,
        ],
    )


def _gin_mlp_body(eps_ref, h0, h1, a0, a1, w1, b1, w2, b2, u_ref, st_ref):
    ep = 1.0 + eps_ref[0, 0]
    z0 = ep * h0[...] + a0[...]
    z1 = ep * h1[...] + a1[...]
    t = jnp.dot(z0, w1[0:HH, :], preferred_element_type=jnp.float32)
    t = t + jnp.dot(z1, w1[HH:D, :], preferred_element_type=jnp.float32)
    t = jnp.maximum(t + b1[...], 0.0)
    u = jnp.dot(t, w2[...], preferred_element_type=jnp.float32) + b2[...]
    u = jnp.maximum(u, 0.0)
    u_ref[...] = u
    stats = jnp.concatenate([jnp.sum(u, axis=0, keepdims=True),
                             jnp.sum(u * u, axis=0, keepdims=True)], axis=0)

    @pl.when(pl.program_id(0) == 0)
    def _init():
        st_ref[...] = stats

    @pl.when(pl.program_id(0) > 0)
    def _acc():
        st_ref[...] += stats


_gin_mlp = pl.pallas_call(
    _gin_mlp_body,
    grid=(NB,),
    in_specs=[
        pl.BlockSpec(memory_space=pltpu.SMEM),
        pl.BlockSpec((R, HH), lambda i: (i, 0)),
        pl.BlockSpec((R, HH), lambda i: (i + NB, 0)),
        pl.BlockSpec((R, HH), lambda i: (i, 0)),
        pl.BlockSpec((R, HH), lambda i: (i + NB, 0)),
        pl.BlockSpec((D, H), lambda i: (0, 0)),
        pl.BlockSpec((1, H), lambda i: (0, 0)),
        pl.BlockSpec((H, H), lambda i: (0, 0)),
        pl.BlockSpec((1, H), lambda i: (0, 0)),
    ],
    out_specs=[
        pl.BlockSpec((R, H), lambda i: (i, 0)),
        pl.BlockSpec((2, H), lambda i: (0, 0)),
    ],
    out_shape=[
        jax.ShapeDtypeStruct((N, H), jnp.float32),
        jax.ShapeDtypeStruct((2, H), jnp.float32),
    ],
)


def _bn_pool_body(u, st, gam, bet, bat, hout_ref, g_ref):
    m = st[0:1, :] * (1.0 / N)
    var = st[1:2, :] * (1.0 / N) - m * m
    a = gam[...] * lax.rsqrt(var + 1e-5)
    b = bet[...] - m * a
    y = u[...] * a + b
    hout_ref[0] = y[:, 0:HH]
    hout_ref[1] = y[:, HH:H]
    seg = lax.broadcasted_iota(jnp.int32, (B, R), 0)
    msk = jnp.where(bat[0] == seg, 1.0, 0.0)
    gp = jnp.dot(msk, y, preferred_element_type=jnp.float32)

    @pl.when(pl.program_id(0) == 0)
    def _init():
        g_ref[...] = gp

    @pl.when(pl.program_id(0) > 0)
    def _acc():
        g_ref[...] += gp


_bn_pool = pl.pallas_call(
    _bn_pool_body,
    grid=(NB,),
    in_specs=[
        pl.BlockSpec((R, H), lambda i: (i, 0)),
        pl.BlockSpec((2, H), lambda i: (0, 0)),
        pl.BlockSpec((1, H), lambda i: (0, 0)),
        pl.BlockSpec((1, H), lambda i: (0, 0)),
        pl.BlockSpec((1, 1, R), lambda i: (i, 0, 0)),
    ],
    out_specs=[
        pl.BlockSpec((2, R, HH), lambda i: (0, i, 0)),
        pl.BlockSpec((B, H), lambda i: (0, 0)),
    ],
    out_shape=[
        jax.ShapeDtypeStruct((2, N, HH), jnp.float32),
        jax.ShapeDtypeStruct((B, H), jnp.float32),
    ],
)


def _cls_body(g1, g2, g3, g4, bat, w1, b1, w2, b2, w3, b3, w4, b4, out_ref):
    seg = lax.broadcasted_iota(jnp.int32, (B, R), 0)
    cnt = jnp.zeros((B, 1), jnp.float32)
    for r in range(NB):
        msk = jnp.where(bat[r:r + 1, :] == seg, 1.0, 0.0)
        cnt = cnt + jnp.sum(msk, axis=1, keepdims=True)
    inv = 1.0 / jnp.maximum(cnt, 1.0)
    z = jnp.dot(g1[...] * inv, w1[0:H, :], preferred_element_type=jnp.float32)
    z = z + jnp.dot(g2[...] * inv, w1[H:2 * H, :], preferred_element_type=jnp.float32)
    z = z + jnp.dot(g3[...] * inv, w1[2 * H:3 * H, :], preferred_element_type=jnp.float32)
    z = z + jnp.dot(g4[...] * inv, w1[3 * H:4 * H, :], preferred_element_type=jnp.float32)
    z = jnp.maximum(z + b1[...], 0.0)
    z = jnp.maximum(jnp.dot(z, w2[...], preferred_element_type=jnp.float32) + b2[...], 0.0)
    z = jnp.maximum(jnp.dot(z, w3[...], preferred_element_type=jnp.float32) + b3[...], 0.0)
    z = jnp.dot(z, w4[...], preferred_element_type=jnp.float32) + b4[...]
    mx = jnp.max(z, axis=1, keepdims=True)
    e = z - mx
    out_ref[...] = e - jnp.log(jnp.sum(jnp.exp(e), axis=1, keepdims=True))


_cls = pl.pallas_call(
    _cls_body,
    grid=(1,),
    in_specs=[
        pl.BlockSpec((B, H), lambda i: (0, 0)),
        pl.BlockSpec((B, H), lambda i: (0, 0)),
        pl.BlockSpec((B, H), lambda i: (0, 0)),
        pl.BlockSpec((B, H), lambda i: (0, 0)),
        pl.BlockSpec((NB, R), lambda i: (0, 0)),
        pl.BlockSpec((4 * H, 2 * H), lambda i: (0, 0)),
        pl.BlockSpec((1, 2 * H), lambda i: (0, 0)),
        pl.BlockSpec((2 * H, H), lambda i: (0, 0)),
        pl.BlockSpec((1, H), lambda i: (0, 0)),
        pl.BlockSpec((H, H), lambda i: (0, 0)),
        pl.BlockSpec((1, H), lambda i: (0, 0)),
        pl.BlockSpec((H, C), lambda i: (0, 0)),
        pl.BlockSpec((1, C), lambda i: (0, 0)),
    ],
    out_specs=pl.BlockSpec((B, C), lambda i: (0, 0)),
    out_shape=jax.ShapeDtypeStruct((B, C), jnp.float32),
)


def kernel(x, edge_index, batch, params):
    src = edge_index[0]
    dst = edge_index[1]
    srcoff = jnp.concatenate([src, src + N]).reshape(2 * NT, CPT, EC)
    dst3d = dst.reshape(NT, CPT, EC)
    zeros_h = jnp.zeros((N, HH), jnp.float32)
    bat2d = batch.reshape(NB, R)
    bat3d = batch.reshape(NB, 1, R)
    h_cat = jnp.concatenate([x[:, :HH], x[:, HH:]], axis=0)

    gs = []
    for p in params['convs']:
        agg = _sc_agg_call()(h_cat, srcoff, dst3d, zeros_h)
        eps = jnp.reshape(p['eps'], (1, 1))
        u, st = _gin_mlp(eps, h_cat, h_cat, agg, agg,
                         p['W1'], p['b1'].reshape(1, H),
                         p['W2'], p['b2'].reshape(1, H))
        hout, g = _bn_pool(u, st, p['gamma'].reshape(1, H),
                           p['beta'].reshape(1, H), bat3d)
        h_cat = hout.reshape(2 * N, HH)
        gs.append(g)

    cl = params['cls']
    return _cls(gs[0], gs[1], gs[2], gs[3], bat2d,
                cl[0]['W'], cl[0]['b'].reshape(1, 2 * H),
                cl[1]['W'], cl[1]['b'].reshape(1, H),
                cl[2]['W'], cl[2]['b'].reshape(1, H),
                cl[3]['W'], cl[3]['b'].reshape(1, C))


# speedup vs baseline: 1.5488x; 1.0722x over previous
import functools

import jax
import jax.numpy as jnp
from jax import lax
from jax.experimental import pallas as pl
from jax.experimental.pallas import tpu as pltpu
from jax.experimental.pallas import tpu_sc as plsc

N = 10000
E = 160000
D = 256
H = 256
HH = 128
B = 64
C = 10
NB = 10
R = N // NB

NSC = 2
NT = 16
EC = 100
EPT = E // NT
CPT = EPT // EC
RPT = N // NT


ZR = 624
ZREM = N - (NT - 1) * ZR - ZR


def _sc_agg_body(h_hbm, src_hbm, dst_hbm, zeros_hbm, out_hbm,
                 sidx_v, didx_v, rows_v, acc, sem, sem2):
    c = lax.axis_index("c")
    s = lax.axis_index("s")
    pltpu.sync_copy(zeros_hbm.at[pl.ds(s * ZR, ZR)], acc.at[pl.ds(s * ZR, ZR)])

    @pl.when(s == NT - 1)
    def _zrem():
        pltpu.sync_copy(zeros_hbm.at[pl.ds(NT * ZR, ZREM)],
                        acc.at[pl.ds(NT * ZR, ZREM)])

    pltpu.sync_copy(src_hbm.at[c * NT + s], sidx_v)
    pltpu.sync_copy(dst_hbm.at[s], didx_v)
    plsc.subcore_barrier()

    def chunk(ci, carry):
        pltpu.async_copy(h_hbm.at[sidx_v.at[ci]], rows_v.at[0], sem).wait()
        pltpu.sync_copy(rows_v.at[0], acc.at[didx_v.at[ci]], add=True)
        return carry

    lax.fori_loop(0, CPT, chunk, 0)
    plsc.subcore_barrier()
    pltpu.sync_copy(acc.at[pl.ds(s * ZR, ZR)],
                    out_hbm.at[pl.ds(c * N + s * ZR, ZR)])

    @pl.when(s == NT - 1)
    def _wrem():
        pltpu.sync_copy(acc.at[pl.ds(NT * ZR, ZREM)],
                        out_hbm.at[pl.ds(c * N + NT * ZR, ZREM)])


@functools.cache
def _sc_agg_call():
    return pl.kernel(
        _sc_agg_body,
        out_type=jax.ShapeDtypeStruct((2 * N, HH), jnp.float32),
        mesh=plsc.VectorSubcoreMesh(core_axis_name="c", subcore_axis_name="s",
                                    num_cores=NSC, num_subcores=NT),
        scratch_types=[
            pltpu.VMEM((CPT, EC), jnp.int32),
            pltpu.VMEM((CPT, EC), jnp.int32),
            pltpu.VMEM((1, EC, HH), jnp.float32),
            pltpu.VMEM_SHARED((N, HH), jnp.float32),
            pltpu.SemaphoreType.DMA,
            pltpu.SemaphoreType.DMA,
        ],
    )


def _gin_mlp_body(eps_ref, h0, h1, a0, a1, w1, b1, w2, b2, u_ref, st_ref):
    ep = 1.0 + eps_ref[0, 0]
    z0 = ep * h0[...] + a0[...]
    z1 = ep * h1[...] + a1[...]
    t = jnp.dot(z0, w1[0:HH, :], preferred_element_type=jnp.float32)
    t = t + jnp.dot(z1, w1[HH:D, :], preferred_element_type=jnp.float32)
    t = jnp.maximum(t + b1[...], 0.0)
    u = jnp.dot(t, w2[...], preferred_element_type=jnp.float32) + b2[...]
    u = jnp.maximum(u, 0.0)
    u_ref[...] = u
    stats = jnp.concatenate([jnp.sum(u, axis=0, keepdims=True),
                             jnp.sum(u * u, axis=0, keepdims=True)], axis=0)

    @pl.when(pl.program_id(0) == 0)
    def _init():
        st_ref[...] = stats

    @pl.when(pl.program_id(0) > 0)
    def _acc():
        st_ref[...] += stats


_gin_mlp = pl.pallas_call(
    _gin_mlp_body,
    grid=(NB,),
    in_specs=[
        pl.BlockSpec(memory_space=pltpu.SMEM),
        pl.BlockSpec((R, HH), lambda i: (i, 0)),
        pl.BlockSpec((R, HH), lambda i: (i + NB, 0)),
        pl.BlockSpec((R, HH), lambda i: (i, 0)),
        pl.BlockSpec((R, HH), lambda i: (i + NB, 0)),
        pl.BlockSpec((D, H), lambda i: (0, 0)),
        pl.BlockSpec((1, H), lambda i: (0, 0)),
        pl.BlockSpec((H, H), lambda i: (0, 0)),
        pl.BlockSpec((1, H), lambda i: (0, 0)),
    ],
    out_specs=[
        pl.BlockSpec((R, H), lambda i: (i, 0)),
        pl.BlockSpec((2, H), lambda i: (0, 0)),
    ],
    out_shape=[
        jax.ShapeDtypeStruct((N, H), jnp.float32),
        jax.ShapeDtypeStruct((2, H), jnp.float32),
    ],
)


def _bn_pool_body(u, st, gam, bet, bat, hout_ref, g_ref):
    m = st[0:1, :] * (1.0 / N)
    var = st[1:2, :] * (1.0 / N) - m * m
    a = gam[...] * lax.rsqrt(var + 1e-5)
    b = bet[...] - m * a
    y = u[...] * a + b
    hout_ref[0] = y[:, 0:HH]
    hout_ref[1] = y[:, HH:H]
    seg = lax.broadcasted_iota(jnp.int32, (B, R), 0)
    msk = jnp.where(bat[0] == seg, 1.0, 0.0)
    gp = jnp.dot(msk, y, preferred_element_type=jnp.float32)

    @pl.when(pl.program_id(0) == 0)
    def _init():
        g_ref[...] = gp

    @pl.when(pl.program_id(0) > 0)
    def _acc():
        g_ref[...] += gp


_bn_pool = pl.pallas_call(
    _bn_pool_body,
    grid=(NB,),
    in_specs=[
        pl.BlockSpec((R, H), lambda i: (i, 0)),
        pl.BlockSpec((2, H), lambda i: (0, 0)),
        pl.BlockSpec((1, H), lambda i: (0, 0)),
        pl.BlockSpec((1, H), lambda i: (0, 0)),
        pl.BlockSpec((1, 1, R), lambda i: (i, 0, 0)),
    ],
    out_specs=[
        pl.BlockSpec((2, R, HH), lambda i: (0, i, 0)),
        pl.BlockSpec((B, H), lambda i: (0, 0)),
    ],
    out_shape=[
        jax.ShapeDtypeStruct((2, N, HH), jnp.float32),
        jax.ShapeDtypeStruct((B, H), jnp.float32),
    ],
)


def _cls_body(g1, g2, g3, g4, bat, w1, b1, w2, b2, w3, b3, w4, b4, out_ref):
    seg = lax.broadcasted_iota(jnp.int32, (B, R), 0)
    cnt = jnp.zeros((B, 1), jnp.float32)
    for r in range(NB):
        msk = jnp.where(bat[r:r + 1, :] == seg, 1.0, 0.0)
        cnt = cnt + jnp.sum(msk, axis=1, keepdims=True)
    inv = 1.0 / jnp.maximum(cnt, 1.0)
    z = jnp.dot(g1[...] * inv, w1[0:H, :], preferred_element_type=jnp.float32)
    z = z + jnp.dot(g2[...] * inv, w1[H:2 * H, :], preferred_element_type=jnp.float32)
    z = z + jnp.dot(g3[...] * inv, w1[2 * H:3 * H, :], preferred_element_type=jnp.float32)
    z = z + jnp.dot(g4[...] * inv, w1[3 * H:4 * H, :], preferred_element_type=jnp.float32)
    z = jnp.maximum(z + b1[...], 0.0)
    z = jnp.maximum(jnp.dot(z, w2[...], preferred_element_type=jnp.float32) + b2[...], 0.0)
    z = jnp.maximum(jnp.dot(z, w3[...], preferred_element_type=jnp.float32) + b3[...], 0.0)
    z = jnp.dot(z, w4[...], preferred_element_type=jnp.float32) + b4[...]
    mx = jnp.max(z, axis=1, keepdims=True)
    e = z - mx
    out_ref[...] = e - jnp.log(jnp.sum(jnp.exp(e), axis=1, keepdims=True))


_cls = pl.pallas_call(
    _cls_body,
    grid=(1,),
    in_specs=[
        pl.BlockSpec((B, H), lambda i: (0, 0)),
        pl.BlockSpec((B, H), lambda i: (0, 0)),
        pl.BlockSpec((B, H), lambda i: (0, 0)),
        pl.BlockSpec((B, H), lambda i: (0, 0)),
        pl.BlockSpec((NB, R), lambda i: (0, 0)),
        pl.BlockSpec((4 * H, 2 * H), lambda i: (0, 0)),
        pl.BlockSpec((1, 2 * H), lambda i: (0, 0)),
        pl.BlockSpec((2 * H, H), lambda i: (0, 0)),
        pl.BlockSpec((1, H), lambda i: (0, 0)),
        pl.BlockSpec((H, H), lambda i: (0, 0)),
        pl.BlockSpec((1, H), lambda i: (0, 0)),
        pl.BlockSpec((H, C), lambda i: (0, 0)),
        pl.BlockSpec((1, C), lambda i: (0, 0)),
    ],
    out_specs=pl.BlockSpec((B, C), lambda i: (0, 0)),
    out_shape=jax.ShapeDtypeStruct((B, C), jnp.float32),
)


def kernel(x, edge_index, batch, params):
    src = edge_index[0]
    dst = edge_index[1]
    srcoff = jnp.concatenate([src, src + N]).reshape(2 * NT, CPT, EC)
    dst3d = dst.reshape(NT, CPT, EC)
    zeros_h = jnp.zeros((N, HH), jnp.float32)
    bat2d = batch.reshape(NB, R)
    bat3d = batch.reshape(NB, 1, R)
    h_cat = jnp.concatenate([x[:, :HH], x[:, HH:]], axis=0)

    gs = []
    for p in params['convs']:
        agg = _sc_agg_call()(h_cat, srcoff, dst3d, zeros_h)
        eps = jnp.reshape(p['eps'], (1, 1))
        u, st = _gin_mlp(eps, h_cat, h_cat, agg, agg,
                         p['W1'], p['b1'].reshape(1, H),
                         p['W2'], p['b2'].reshape(1, H))
        hout, g = _bn_pool(u, st, p['gamma'].reshape(1, H),
                           p['beta'].reshape(1, H), bat3d)
        h_cat = hout.reshape(2 * N, HH)
        gs.append(g)

    cl = params['cls']
    return _cls(gs[0], gs[1], gs[2], gs[3], bat2d,
                cl[0]['W'], cl[0]['b'].reshape(1, 2 * H),
                cl[1]['W'], cl[1]['b'].reshape(1, H),
                cl[2]['W'], cl[2]['b'].reshape(1, H),
                cl[3]['W'], cl[3]['b'].reshape(1, C))


# speedup vs baseline: 1.6507x; 1.0658x over previous
import functools

import jax
import jax.numpy as jnp
from jax import lax
from jax.experimental import pallas as pl
from jax.experimental.pallas import tpu as pltpu
from jax.experimental.pallas import tpu_sc as plsc

N = 10000
E = 160000
D = 256
H = 256
HH = 128
B = 64
C = 10
NB = 10
R = N // NB

NSC = 2
NT = 16
EC = 125
EPT = E // NT
CPT = EPT // EC
RPT = N // NT


ZR = 624
ZREM = N - (NT - 1) * ZR - ZR


def _sc_agg_body(h_hbm, src_hbm, dst_hbm, zeros_hbm, out_hbm,
                 sidx_v, didx_v, rows_v, acc, sem, sem2):
    c = lax.axis_index("c")
    s = lax.axis_index("s")
    pltpu.sync_copy(zeros_hbm.at[pl.ds(s * ZR, ZR)], acc.at[pl.ds(s * ZR, ZR)])

    @pl.when(s == NT - 1)
    def _zrem():
        pltpu.sync_copy(zeros_hbm.at[pl.ds(NT * ZR, ZREM)],
                        acc.at[pl.ds(NT * ZR, ZREM)])

    pltpu.sync_copy(src_hbm.at[c * NT + s], sidx_v)
    pltpu.sync_copy(dst_hbm.at[s], didx_v)
    plsc.subcore_barrier()

    def chunk(ci, carry):
        pltpu.async_copy(h_hbm.at[sidx_v.at[ci]], rows_v.at[0], sem).wait()
        pltpu.sync_copy(rows_v.at[0], acc.at[didx_v.at[ci]], add=True)
        return carry

    lax.fori_loop(0, CPT, chunk, 0)
    plsc.subcore_barrier()
    pltpu.sync_copy(acc.at[pl.ds(s * ZR, ZR)],
                    out_hbm.at[pl.ds(c * N + s * ZR, ZR)])

    @pl.when(s == NT - 1)
    def _wrem():
        pltpu.sync_copy(acc.at[pl.ds(NT * ZR, ZREM)],
                        out_hbm.at[pl.ds(c * N + NT * ZR, ZREM)])


@functools.cache
def _sc_agg_call():
    return pl.kernel(
        _sc_agg_body,
        out_type=jax.ShapeDtypeStruct((2 * N, HH), jnp.float32),
        mesh=plsc.VectorSubcoreMesh(core_axis_name="c", subcore_axis_name="s",
                                    num_cores=NSC, num_subcores=NT),
        scratch_types=[
            pltpu.VMEM((CPT, EC), jnp.int32),
            pltpu.VMEM((CPT, EC), jnp.int32),
            pltpu.VMEM((1, EC, HH), jnp.float32),
            pltpu.VMEM_SHARED((N, HH), jnp.float32),
            pltpu.SemaphoreType.DMA,
            pltpu.SemaphoreType.DMA,
        ],
    )


def _gin_mlp_body(eps_ref, h0, h1, a0, a1, w1, b1, w2, b2, u_ref, st_ref):
    ep = 1.0 + eps_ref[0, 0]
    z0 = ep * h0[...] + a0[...]
    z1 = ep * h1[...] + a1[...]
    t = jnp.dot(z0, w1[0:HH, :], preferred_element_type=jnp.float32)
    t = t + jnp.dot(z1, w1[HH:D, :], preferred_element_type=jnp.float32)
    t = jnp.maximum(t + b1[...], 0.0)
    u = jnp.dot(t, w2[...], preferred_element_type=jnp.float32) + b2[...]
    u = jnp.maximum(u, 0.0)
    u_ref[...] = u
    stats = jnp.concatenate([jnp.sum(u, axis=0, keepdims=True),
                             jnp.sum(u * u, axis=0, keepdims=True)], axis=0)

    @pl.when(pl.program_id(0) == 0)
    def _init():
        st_ref[...] = stats

    @pl.when(pl.program_id(0) > 0)
    def _acc():
        st_ref[...] += stats


_gin_mlp = pl.pallas_call(
    _gin_mlp_body,
    grid=(NB,),
    in_specs=[
        pl.BlockSpec(memory_space=pltpu.SMEM),
        pl.BlockSpec((R, HH), lambda i: (i, 0)),
        pl.BlockSpec((R, HH), lambda i: (i + NB, 0)),
        pl.BlockSpec((R, HH), lambda i: (i, 0)),
        pl.BlockSpec((R, HH), lambda i: (i + NB, 0)),
        pl.BlockSpec((D, H), lambda i: (0, 0)),
        pl.BlockSpec((1, H), lambda i: (0, 0)),
        pl.BlockSpec((H, H), lambda i: (0, 0)),
        pl.BlockSpec((1, H), lambda i: (0, 0)),
    ],
    out_specs=[
        pl.BlockSpec((R, H), lambda i: (i, 0)),
        pl.BlockSpec((2, H), lambda i: (0, 0)),
    ],
    out_shape=[
        jax.ShapeDtypeStruct((N, H), jnp.float32),
        jax.ShapeDtypeStruct((2, H), jnp.float32),
    ],
)


def _bn_pool_body(u, st, gam, bet, bat, hout_ref, g_ref):
    m = st[0:1, :] * (1.0 / N)
    var = st[1:2, :] * (1.0 / N) - m * m
    a = gam[...] * lax.rsqrt(var + 1e-5)
    b = bet[...] - m * a
    y = u[...] * a + b
    hout_ref[0] = y[:, 0:HH]
    hout_ref[1] = y[:, HH:H]
    seg = lax.broadcasted_iota(jnp.int32, (B, R), 0)
    msk = jnp.where(bat[0] == seg, 1.0, 0.0)
    gp = jnp.dot(msk, y, preferred_element_type=jnp.float32)

    @pl.when(pl.program_id(0) == 0)
    def _init():
        g_ref[...] = gp

    @pl.when(pl.program_id(0) > 0)
    def _acc():
        g_ref[...] += gp


_bn_pool = pl.pallas_call(
    _bn_pool_body,
    grid=(NB,),
    in_specs=[
        pl.BlockSpec((R, H), lambda i: (i, 0)),
        pl.BlockSpec((2, H), lambda i: (0, 0)),
        pl.BlockSpec((1, H), lambda i: (0, 0)),
        pl.BlockSpec((1, H), lambda i: (0, 0)),
        pl.BlockSpec((1, 1, R), lambda i: (i, 0, 0)),
    ],
    out_specs=[
        pl.BlockSpec((2, R, HH), lambda i: (0, i, 0)),
        pl.BlockSpec((B, H), lambda i: (0, 0)),
    ],
    out_shape=[
        jax.ShapeDtypeStruct((2, N, HH), jnp.float32),
        jax.ShapeDtypeStruct((B, H), jnp.float32),
    ],
)


def _cls_body(g1, g2, g3, g4, bat, w1, b1, w2, b2, w3, b3, w4, b4, out_ref):
    seg = lax.broadcasted_iota(jnp.int32, (B, R), 0)
    cnt = jnp.zeros((B, 1), jnp.float32)
    for r in range(NB):
        msk = jnp.where(bat[r:r + 1, :] == seg, 1.0, 0.0)
        cnt = cnt + jnp.sum(msk, axis=1, keepdims=True)
    inv = 1.0 / jnp.maximum(cnt, 1.0)
    z = jnp.dot(g1[...] * inv, w1[0:H, :], preferred_element_type=jnp.float32)
    z = z + jnp.dot(g2[...] * inv, w1[H:2 * H, :], preferred_element_type=jnp.float32)
    z = z + jnp.dot(g3[...] * inv, w1[2 * H:3 * H, :], preferred_element_type=jnp.float32)
    z = z + jnp.dot(g4[...] * inv, w1[3 * H:4 * H, :], preferred_element_type=jnp.float32)
    z = jnp.maximum(z + b1[...], 0.0)
    z = jnp.maximum(jnp.dot(z, w2[...], preferred_element_type=jnp.float32) + b2[...], 0.0)
    z = jnp.maximum(jnp.dot(z, w3[...], preferred_element_type=jnp.float32) + b3[...], 0.0)
    z = jnp.dot(z, w4[...], preferred_element_type=jnp.float32) + b4[...]
    mx = jnp.max(z, axis=1, keepdims=True)
    e = z - mx
    out_ref[...] = e - jnp.log(jnp.sum(jnp.exp(e), axis=1, keepdims=True))


_cls = pl.pallas_call(
    _cls_body,
    grid=(1,),
    in_specs=[
        pl.BlockSpec((B, H), lambda i: (0, 0)),
        pl.BlockSpec((B, H), lambda i: (0, 0)),
        pl.BlockSpec((B, H), lambda i: (0, 0)),
        pl.BlockSpec((B, H), lambda i: (0, 0)),
        pl.BlockSpec((NB, R), lambda i: (0, 0)),
        pl.BlockSpec((4 * H, 2 * H), lambda i: (0, 0)),
        pl.BlockSpec((1, 2 * H), lambda i: (0, 0)),
        pl.BlockSpec((2 * H, H), lambda i: (0, 0)),
        pl.BlockSpec((1, H), lambda i: (0, 0)),
        pl.BlockSpec((H, H), lambda i: (0, 0)),
        pl.BlockSpec((1, H), lambda i: (0, 0)),
        pl.BlockSpec((H, C), lambda i: (0, 0)),
        pl.BlockSpec((1, C), lambda i: (0, 0)),
    ],
    out_specs=pl.BlockSpec((B, C), lambda i: (0, 0)),
    out_shape=jax.ShapeDtypeStruct((B, C), jnp.float32),
)


def kernel(x, edge_index, batch, params):
    src = edge_index[0]
    dst = edge_index[1]
    srcoff = jnp.concatenate([src, src + N]).reshape(2 * NT, CPT, EC)
    dst3d = dst.reshape(NT, CPT, EC)
    zeros_h = jnp.zeros((N, HH), jnp.float32)
    bat2d = batch.reshape(NB, R)
    bat3d = batch.reshape(NB, 1, R)
    h_cat = jnp.concatenate([x[:, :HH], x[:, HH:]], axis=0)

    gs = []
    for p in params['convs']:
        agg = _sc_agg_call()(h_cat, srcoff, dst3d, zeros_h)
        eps = jnp.reshape(p['eps'], (1, 1))
        u, st = _gin_mlp(eps, h_cat, h_cat, agg, agg,
                         p['W1'], p['b1'].reshape(1, H),
                         p['W2'], p['b2'].reshape(1, H))
        hout, g = _bn_pool(u, st, p['gamma'].reshape(1, H),
                           p['beta'].reshape(1, H), bat3d)
        h_cat = hout.reshape(2 * N, HH)
        gs.append(g)

    cl = params['cls']
    return _cls(gs[0], gs[1], gs[2], gs[3], bat2d,
                cl[0]['W'], cl[0]['b'].reshape(1, 2 * H),
                cl[1]['W'], cl[1]['b'].reshape(1, H),
                cl[2]['W'], cl[2]['b'].reshape(1, H),
                cl[3]['W'], cl[3]['b'].reshape(1, C))


# speedup vs baseline: 1.7042x; 1.0324x over previous
import functools

import jax
import jax.numpy as jnp
from jax import lax
from jax.experimental import pallas as pl
from jax.experimental.pallas import tpu as pltpu
from jax.experimental.pallas import tpu_sc as plsc

N = 10000
E = 160000
D = 256
H = 256
HH = 128
B = 64
C = 10
NB = 10
R = N // NB

NSC = 2
NT = 16
EC = 125
EPT = E // NT
CPT = EPT // EC
RPT = N // NT


ZR = 624
ZREM = N - (NT - 1) * ZR - ZR


def _sc_agg_body(h_hbm, src_hbm, dst_hbm, zeros_hbm, out_hbm,
                 sidx_v, didx_v, rows_v, acc, sem, sem2):
    c = lax.axis_index("c")
    s = lax.axis_index("s")
    pltpu.sync_copy(zeros_hbm.at[pl.ds(s * ZR, ZR)], acc.at[pl.ds(s * ZR, ZR)])

    @pl.when(s == NT - 1)
    def _zrem():
        pltpu.sync_copy(zeros_hbm.at[pl.ds(NT * ZR, ZREM)],
                        acc.at[pl.ds(NT * ZR, ZREM)])

    pltpu.sync_copy(src_hbm.at[c * NT + s], sidx_v)
    pltpu.sync_copy(dst_hbm.at[s], didx_v)
    plsc.subcore_barrier()

    def chunk(ci, carry):
        pltpu.async_copy(h_hbm.at[sidx_v.at[ci]], rows_v.at[0], sem).wait()
        pltpu.sync_copy(rows_v.at[0], acc.at[didx_v.at[ci]], add=True)
        return carry

    lax.fori_loop(0, CPT, chunk, 0)
    plsc.subcore_barrier()
    pltpu.sync_copy(acc.at[pl.ds(s * ZR, ZR)],
                    out_hbm.at[pl.ds(c * N + s * ZR, ZR)])

    @pl.when(s == NT - 1)
    def _wrem():
        pltpu.sync_copy(acc.at[pl.ds(NT * ZR, ZREM)],
                        out_hbm.at[pl.ds(c * N + NT * ZR, ZREM)])


@functools.cache
def _sc_agg_call():
    return pl.kernel(
        _sc_agg_body,
        out_type=jax.ShapeDtypeStruct((2 * N, HH), jnp.float32),
        mesh=plsc.VectorSubcoreMesh(core_axis_name="c", subcore_axis_name="s",
                                    num_cores=NSC, num_subcores=NT),
        scratch_types=[
            pltpu.VMEM((CPT, EC), jnp.int32),
            pltpu.VMEM((CPT, EC), jnp.int32),
            pltpu.VMEM((1, EC, HH), jnp.float32),
            pltpu.VMEM_SHARED((N, HH), jnp.float32),
            pltpu.SemaphoreType.DMA,
            pltpu.SemaphoreType.DMA,
        ],
    )


def _gin_fused_body(eps_ref, h0, h1, a0, a1, w1, b1, w2, b2, gam, bet, bat,
                    hout_ref, g_ref, u_scr, st_scr):
    p = pl.program_id(0)
    i = pl.program_id(1)

    @pl.when(p == 0)
    def _compute():
        ep = 1.0 + eps_ref[0, 0]
        z0 = ep * h0[...] + a0[...]
        z1 = ep * h1[...] + a1[...]
        t = jnp.dot(z0, w1[0:HH, :], preferred_element_type=jnp.float32)
        t = t + jnp.dot(z1, w1[HH:D, :], preferred_element_type=jnp.float32)
        t = jnp.maximum(t + b1[...], 0.0)
        u = jnp.dot(t, w2[...], preferred_element_type=jnp.float32) + b2[...]
        u = jnp.maximum(u, 0.0)
        u_scr[pl.ds(i * R, R), :] = u
        stats = jnp.concatenate([jnp.sum(u, axis=0, keepdims=True),
                                 jnp.sum(u * u, axis=0, keepdims=True)], axis=0)

        @pl.when(i == 0)
        def _init():
            st_scr[...] = stats

        @pl.when(i > 0)
        def _acc():
            st_scr[...] += stats

    @pl.when(p == 1)
    def _norm():
        m = st_scr[0:1, :] * (1.0 / N)
        var = st_scr[1:2, :] * (1.0 / N) - m * m
        a = gam[...] * lax.rsqrt(var + 1e-5)
        b = bet[...] - m * a
        y = u_scr[pl.ds(i * R, R), :] * a + b
        hout_ref[0] = y[:, 0:HH]
        hout_ref[1] = y[:, HH:H]
        seg = lax.broadcasted_iota(jnp.int32, (B, R), 0)
        msk = jnp.where(bat[0] == seg, 1.0, 0.0)
        gp = jnp.dot(msk, y, preferred_element_type=jnp.float32)

        @pl.when(i == 0)
        def _init():
            g_ref[...] = gp

        @pl.when(i > 0)
        def _acc():
            g_ref[...] += gp


_gin_fused = pl.pallas_call(
    _gin_fused_body,
    grid=(2, NB),
    in_specs=[
        pl.BlockSpec(memory_space=pltpu.SMEM),
        pl.BlockSpec((R, HH), lambda p, i: (jnp.where(p == 0, i, 0), 0)),
        pl.BlockSpec((R, HH), lambda p, i: (jnp.where(p == 0, i + NB, 0), 0)),
        pl.BlockSpec((R, HH), lambda p, i: (jnp.where(p == 0, i, 0), 0)),
        pl.BlockSpec((R, HH), lambda p, i: (jnp.where(p == 0, i + NB, 0), 0)),
        pl.BlockSpec((D, H), lambda p, i: (0, 0)),
        pl.BlockSpec((1, H), lambda p, i: (0, 0)),
        pl.BlockSpec((H, H), lambda p, i: (0, 0)),
        pl.BlockSpec((1, H), lambda p, i: (0, 0)),
        pl.BlockSpec((1, H), lambda p, i: (0, 0)),
        pl.BlockSpec((1, H), lambda p, i: (0, 0)),
        pl.BlockSpec((1, 1, R), lambda p, i: (jnp.where(p == 0, 0, i), 0, 0)),
    ],
    out_specs=[
        pl.BlockSpec((2, R, HH), lambda p, i: (0, jnp.where(p == 0, 0, i), 0)),
        pl.BlockSpec((B, H), lambda p, i: (0, 0)),
    ],
    out_shape=[
        jax.ShapeDtypeStruct((2, N, HH), jnp.float32),
        jax.ShapeDtypeStruct((B, H), jnp.float32),
    ],
    scratch_shapes=[
        pltpu.VMEM((N, H), jnp.float32),
        pltpu.VMEM((2, H), jnp.float32),
    ],
)


def _cls_body(g1, g2, g3, g4, bat, w1, b1, w2, b2, w3, b3, w4, b4, out_ref):
    seg = lax.broadcasted_iota(jnp.int32, (B, R), 0)
    cnt = jnp.zeros((B, 1), jnp.float32)
    for r in range(NB):
        msk = jnp.where(bat[r:r + 1, :] == seg, 1.0, 0.0)
        cnt = cnt + jnp.sum(msk, axis=1, keepdims=True)
    inv = 1.0 / jnp.maximum(cnt, 1.0)
    z = jnp.dot(g1[...] * inv, w1[0:H, :], preferred_element_type=jnp.float32)
    z = z + jnp.dot(g2[...] * inv, w1[H:2 * H, :], preferred_element_type=jnp.float32)
    z = z + jnp.dot(g3[...] * inv, w1[2 * H:3 * H, :], preferred_element_type=jnp.float32)
    z = z + jnp.dot(g4[...] * inv, w1[3 * H:4 * H, :], preferred_element_type=jnp.float32)
    z = jnp.maximum(z + b1[...], 0.0)
    z = jnp.maximum(jnp.dot(z, w2[...], preferred_element_type=jnp.float32) + b2[...], 0.0)
    z = jnp.maximum(jnp.dot(z, w3[...], preferred_element_type=jnp.float32) + b3[...], 0.0)
    z = jnp.dot(z, w4[...], preferred_element_type=jnp.float32) + b4[...]
    mx = jnp.max(z, axis=1, keepdims=True)
    e = z - mx
    out_ref[...] = e - jnp.log(jnp.sum(jnp.exp(e), axis=1, keepdims=True))


_cls = pl.pallas_call(
    _cls_body,
    grid=(1,),
    in_specs=[
        pl.BlockSpec((B, H), lambda i: (0, 0)),
        pl.BlockSpec((B, H), lambda i: (0, 0)),
        pl.BlockSpec((B, H), lambda i: (0, 0)),
        pl.BlockSpec((B, H), lambda i: (0, 0)),
        pl.BlockSpec((NB, R), lambda i: (0, 0)),
        pl.BlockSpec((4 * H, 2 * H), lambda i: (0, 0)),
        pl.BlockSpec((1, 2 * H), lambda i: (0, 0)),
        pl.BlockSpec((2 * H, H), lambda i: (0, 0)),
        pl.BlockSpec((1, H), lambda i: (0, 0)),
        pl.BlockSpec((H, H), lambda i: (0, 0)),
        pl.BlockSpec((1, H), lambda i: (0, 0)),
        pl.BlockSpec((H, C), lambda i: (0, 0)),
        pl.BlockSpec((1, C), lambda i: (0, 0)),
    ],
    out_specs=pl.BlockSpec((B, C), lambda i: (0, 0)),
    out_shape=jax.ShapeDtypeStruct((B, C), jnp.float32),
)


def kernel(x, edge_index, batch, params):
    src = edge_index[0]
    dst = edge_index[1]
    srcoff = jnp.concatenate([src, src + N]).reshape(2 * NT, CPT, EC)
    dst3d = dst.reshape(NT, CPT, EC)
    zeros_h = jnp.zeros((N, HH), jnp.float32)
    bat2d = batch.reshape(NB, R)
    bat3d = batch.reshape(NB, 1, R)
    h_cat = jnp.concatenate([x[:, :HH], x[:, HH:]], axis=0)

    gs = []
    for p in params['convs']:
        agg = _sc_agg_call()(h_cat, srcoff, dst3d, zeros_h)
        eps = jnp.reshape(p['eps'], (1, 1))
        hout, g = _gin_fused(eps, h_cat, h_cat, agg, agg,
                             p['W1'], p['b1'].reshape(1, H),
                             p['W2'], p['b2'].reshape(1, H),
                             p['gamma'].reshape(1, H),
                             p['beta'].reshape(1, H), bat3d)
        h_cat = hout.reshape(2 * N, HH)
        gs.append(g)

    cl = params['cls']
    return _cls(gs[0], gs[1], gs[2], gs[3], bat2d,
                cl[0]['W'], cl[0]['b'].reshape(1, 2 * H),
                cl[1]['W'], cl[1]['b'].reshape(1, H),
                cl[2]['W'], cl[2]['b'].reshape(1, H),
                cl[3]['W'], cl[3]['b'].reshape(1, C))
